# baseline (speedup 1.0000x reference)
import jax
import jax.numpy as jnp
from jax import lax
from jax.experimental import pallas as pl
from jax.experimental.pallas import tpu as pltpu
from jax.experimental.pallas import tpu_sc as plsc

N_NODES = 10000
N_EDGES = 320000
D_FEAT = 128
NCLS = 16
DOUT = 2 * NCLS

NC = 2
NS = 16
NW = NC * NS
L = 16

PER_W = N_EDGES // NW
G = 80
CSUB = 5
CH = G * CSUB
NCHUNK = PER_W // CH
NPAIR = (NCHUNK + 1) // 2
ROWS_E = N_EDGES // 8
ROWS_CH = CH * NCLS // 128


def _proj_body(f_ref, ww_ref, wx_ref, o_ref):
    o_ref[...] = jnp.concatenate(
        [jnp.dot(f_ref[...], ww_ref[...].T, preferred_element_type=jnp.float32),
         jnp.dot(f_ref[...], wx_ref[...].T, preferred_element_type=jnp.float32)],
        axis=1)


_proj = pl.pallas_call(
    _proj_body,
    out_shape=jax.ShapeDtypeStruct((N_NODES, DOUT), jnp.float32),
)


def _edge_body(p_hbm, edge_hbm, bw_hbm, bx_hbm, out_hbm,
               idx_s, idx_d, rows_sa, rows_da, rows_sb, rows_db,
               out_w, out_x, bias_v, sem_a, sem_b):
    wid = lax.axis_index("s") * NC + lax.axis_index("c")
    wbase = pl.multiple_of(wid * PER_W, 8)
    pltpu.sync_copy(bw_hbm, bias_v.at[pl.ds(0, NCLS)])
    pltpu.sync_copy(bx_hbm, bias_v.at[pl.ds(NCLS, NCLS)])
    pltpu.sync_copy(edge_hbm.at[0, pl.ds(wbase, PER_W)], idx_s)
    pltpu.sync_copy(edge_hbm.at[1, pl.ds(wbase, PER_W)], idx_d)
    b_lo = bias_v[pl.ds(0, L)]
    b_hi = bias_v[pl.ds(L, L)]

    def fire(c, rows_s, rows_d, sem):
        cbase = pl.multiple_of(c * CH, 8)
        for j in range(CSUB):
            pltpu.async_copy(p_hbm.at[idx_s.at[pl.ds(cbase + j * G, G)]],
                             rows_s.at[pl.ds(j * G, G)], sem)
            pltpu.async_copy(p_hbm.at[idx_d.at[pl.ds(cbase + j * G, G)]],
                             rows_d.at[pl.ds(j * G, G)], sem)

    def drain(rows_s, rows_d, sem):
        for j in range(CSUB):
            pltpu.make_async_copy(p_hbm.at[pl.ds(0, G)],
                                  rows_s.at[pl.ds(j * G, G)], sem).wait()
            pltpu.make_async_copy(p_hbm.at[pl.ds(0, G)],
                                  rows_d.at[pl.ds(j * G, G)], sem).wait()

    def run_chunk(c, rows_s, rows_d):
        @plsc.parallel_loop(0, ROWS_CH, step=1, unroll=2)
        def _(i8):
            for k in range(8):
                e = i8 * 8 + k
                out_w[i8, pl.ds(k * L, L)] = (rows_s[e, pl.ds(0, L)]
                                              - rows_d[e, pl.ds(0, L)] + b_lo)
                out_x[i8, pl.ds(k * L, L)] = (rows_s[e, pl.ds(L, L)]
                                              - rows_d[e, pl.ds(L, L)] + b_hi)
        rbase = (wbase + c * CH) // 8
        pltpu.sync_copy(out_w, out_hbm.at[0, pl.ds(rbase, ROWS_CH)])
        pltpu.sync_copy(out_x, out_hbm.at[1, pl.ds(rbase, ROWS_CH)])

    fire(0, rows_sa, rows_da, sem_a)

    def pair_body(t, carry):
        c0 = 2 * t
        c1 = c0 + 1

        @pl.when(c1 < NCHUNK)
        def _():
            fire(c1, rows_sb, rows_db, sem_b)

        drain(rows_sa, rows_da, sem_a)
        run_chunk(c0, rows_sa, rows_da)

        @pl.when(c1 < NCHUNK)
        def _():
            @pl.when(c1 + 1 < NCHUNK)
            def _():
                fire(c1 + 1, rows_sa, rows_da, sem_a)

            drain(rows_sb, rows_db, sem_b)
            run_chunk(c1, rows_sb, rows_db)

        return carry

    lax.fori_loop(0, NPAIR, pair_body, 0)


_edge = pl.kernel(
    _edge_body,
    out_type=jax.ShapeDtypeStruct((2, ROWS_E, 128), jnp.float32),
    mesh=plsc.VectorSubcoreMesh(core_axis_name="c", subcore_axis_name="s",
                                num_cores=NC, num_subcores=NS),
    compiler_params=pltpu.CompilerParams(use_tc_tiling_on_sc=False),
    scratch_types=[
        pltpu.VMEM((PER_W,), jnp.int32),
        pltpu.VMEM((PER_W,), jnp.int32),
        pltpu.VMEM((CH, DOUT), jnp.float32),
        pltpu.VMEM((CH, DOUT), jnp.float32),
        pltpu.VMEM((CH, DOUT), jnp.float32),
        pltpu.VMEM((CH, DOUT), jnp.float32),
        pltpu.VMEM((ROWS_CH, 128), jnp.float32),
        pltpu.VMEM((ROWS_CH, 128), jnp.float32),
        pltpu.VMEM((DOUT,), jnp.float32),
        pltpu.SemaphoreType.DMA,
        pltpu.SemaphoreType.DMA,
    ],
)

_UNPACK_BM = 2000
_UNPACK_NBLK = ROWS_E // _UNPACK_BM


def _unpack_body(a_ref, w_ref, x_ref):
    w_ref[...] = a_ref[0].reshape(_UNPACK_BM * 8, NCLS)
    x_ref[...] = a_ref[1].reshape(_UNPACK_BM * 8, NCLS)


_unpack = pl.pallas_call(
    _unpack_body,
    grid=(_UNPACK_NBLK,),
    in_specs=[pl.BlockSpec((2, _UNPACK_BM, 128), lambda i: (0, i, 0))],
    out_specs=[pl.BlockSpec((_UNPACK_BM * 8, NCLS), lambda i: (i, 0)),
               pl.BlockSpec((_UNPACK_BM * 8, NCLS), lambda i: (i, 0))],
    out_shape=(jax.ShapeDtypeStruct((N_EDGES, NCLS), jnp.float32),
               jax.ShapeDtypeStruct((N_EDGES, NCLS), jnp.float32)),
)


def kernel(features, edge_index, W_w, b_w, W_x, b_x):
    p = _proj(features, W_w, W_x)
    ei = edge_index.astype(jnp.int32)
    packed = _edge(p, ei, b_w, b_x)
    w = packed[0].reshape(N_EDGES, NCLS)
    x = packed[1].reshape(N_EDGES, NCLS)
    return w, x


# speedup vs baseline: 1.0853x; 1.0853x over previous
import jax
import jax.numpy as jnp
from jax import lax
from jax.experimental import pallas as pl
from jax.experimental.pallas import tpu as pltpu
from jax.experimental.pallas import tpu_sc as plsc

N_NODES = 10000
N_EDGES = 320000
D_FEAT = 128
NCLS = 16
DOUT = 2 * NCLS

NC = 2
NS = 16
NW = NC * NS
L = 16

PER_W = N_EDGES // NW
G = 80
CSUB = 5
CH = G * CSUB
NCHUNK = PER_W // CH
NPAIR = (NCHUNK + 1) // 2
ROWS_E = N_EDGES // 8
ROWS_CH = CH * NCLS // 128


def _proj_body(f_ref, ww_ref, wx_ref, o_ref):
    o_ref[...] = jnp.concatenate(
        [jnp.dot(f_ref[...], ww_ref[...].T, preferred_element_type=jnp.float32),
         jnp.dot(f_ref[...], wx_ref[...].T, preferred_element_type=jnp.float32)],
        axis=1)


_proj = pl.pallas_call(
    _proj_body,
    out_shape=jax.ShapeDtypeStruct((N_NODES, DOUT), jnp.float32),
)


def _edge_body(p_hbm, edge_hbm, bw_hbm, bx_hbm, pw_hbm, px_hbm,
               idx_s, idx_d, rows_sa, rows_da, rows_sb, rows_db,
               out_w, out_x, bias_v, sem_a, sem_b):
    wid = lax.axis_index("s") * NC + lax.axis_index("c")
    wbase = pl.multiple_of(wid * PER_W, 8)
    pltpu.sync_copy(bw_hbm, bias_v.at[pl.ds(0, NCLS)])
    pltpu.sync_copy(bx_hbm, bias_v.at[pl.ds(NCLS, NCLS)])
    pltpu.sync_copy(edge_hbm.at[0, pl.ds(wbase, PER_W)], idx_s)
    pltpu.sync_copy(edge_hbm.at[1, pl.ds(wbase, PER_W)], idx_d)
    b_lo = bias_v[pl.ds(0, L)]
    b_hi = bias_v[pl.ds(L, L)]

    def fire(c, rows_s, rows_d, sem):
        cbase = pl.multiple_of(c * CH, 8)
        for j in range(CSUB):
            pltpu.async_copy(p_hbm.at[idx_s.at[pl.ds(cbase + j * G, G)]],
                             rows_s.at[pl.ds(j * G, G)], sem)
            pltpu.async_copy(p_hbm.at[idx_d.at[pl.ds(cbase + j * G, G)]],
                             rows_d.at[pl.ds(j * G, G)], sem)

    def drain(rows_s, rows_d, sem):
        for j in range(CSUB):
            pltpu.make_async_copy(p_hbm.at[pl.ds(0, G)],
                                  rows_s.at[pl.ds(j * G, G)], sem).wait()
            pltpu.make_async_copy(p_hbm.at[pl.ds(0, G)],
                                  rows_d.at[pl.ds(j * G, G)], sem).wait()

    def run_chunk(c, rows_s, rows_d):
        @plsc.parallel_loop(0, CH, step=1, unroll=8)
        def _(i):
            out_w[i, :] = (rows_s[i, pl.ds(0, L)]
                           - rows_d[i, pl.ds(0, L)] + b_lo)
            out_x[i, :] = (rows_s[i, pl.ds(L, L)]
                           - rows_d[i, pl.ds(L, L)] + b_hi)
        ebase = pl.multiple_of(wbase + c * CH, 8)
        pltpu.sync_copy(out_w, pw_hbm.at[pl.ds(ebase, CH)])
        pltpu.sync_copy(out_x, px_hbm.at[pl.ds(ebase, CH)])

    fire(0, rows_sa, rows_da, sem_a)

    def pair_body(t, carry):
        c0 = 2 * t
        c1 = c0 + 1

        @pl.when(c1 < NCHUNK)
        def _():
            fire(c1, rows_sb, rows_db, sem_b)

        drain(rows_sa, rows_da, sem_a)
        run_chunk(c0, rows_sa, rows_da)

        @pl.when(c1 < NCHUNK)
        def _():
            @pl.when(c1 + 1 < NCHUNK)
            def _():
                fire(c1 + 1, rows_sa, rows_da, sem_a)

            drain(rows_sb, rows_db, sem_b)
            run_chunk(c1, rows_sb, rows_db)

        return carry

    lax.fori_loop(0, NPAIR, pair_body, 0)


_edge = pl.kernel(
    _edge_body,
    out_type=(jax.ShapeDtypeStruct((N_EDGES, NCLS), jnp.float32),
              jax.ShapeDtypeStruct((N_EDGES, NCLS), jnp.float32)),
    mesh=plsc.VectorSubcoreMesh(core_axis_name="c", subcore_axis_name="s",
                                num_cores=NC, num_subcores=NS),
    compiler_params=pltpu.CompilerParams(use_tc_tiling_on_sc=False),
    scratch_types=[
        pltpu.VMEM((PER_W,), jnp.int32),
        pltpu.VMEM((PER_W,), jnp.int32),
        pltpu.VMEM((CH, DOUT), jnp.float32),
        pltpu.VMEM((CH, DOUT), jnp.float32),
        pltpu.VMEM((CH, DOUT), jnp.float32),
        pltpu.VMEM((CH, DOUT), jnp.float32),
        pltpu.VMEM((CH, NCLS), jnp.float32),
        pltpu.VMEM((CH, NCLS), jnp.float32),
        pltpu.VMEM((DOUT,), jnp.float32),
        pltpu.SemaphoreType.DMA,
        pltpu.SemaphoreType.DMA,
    ],
)

def kernel(features, edge_index, W_w, b_w, W_x, b_x):
    p = _proj(features, W_w, W_x)
    ei = edge_index.astype(jnp.int32)
    w, x = _edge(p, ei, b_w, b_x)
    return w, x
